# baseline (speedup 1.0000x reference)
import functools

import jax
import jax.numpy as jnp
from jax import lax
from jax.experimental import pallas as pl
from jax.experimental.pallas import tpu as pltpu
from jax.experimental.pallas import tpu_sc as plsc

N_BINS = 256
N_BOUNDS = 255
D = 512
ROWS_BLOCK = 1024


def _sc_bucketize_body(et_hbm, pt_hbm, eb_hbm, pb_hbm, ie_hbm, ip_hbm,
                       te_v, tp_v, ide_v, idp_v, eb_v, pb_v,
                       *, rows_per_w, num_cores):
    wid = lax.axis_index("s") * num_cores + lax.axis_index("c")
    base = wid * rows_per_w
    pltpu.sync_copy(eb_hbm, eb_v)
    pltpu.sync_copy(pb_hbm, pb_v)
    pltpu.sync_copy(et_hbm.at[pl.ds(base, rows_per_w)], te_v)
    pltpu.sync_copy(pt_hbm.at[pl.ds(base, rows_per_w)], tp_v)

    unroll = 2
    n_iter = rows_per_w // (16 * unroll)

    def body(i, _):
        for u in range(unroll):
            off = (i * unroll + u) * 16
            ve = te_v[pl.ds(off, 16)]
            vp = tp_v[pl.ds(off, 16)]
            ce = jnp.zeros((16,), jnp.int32)
            cp = jnp.zeros((16,), jnp.int32)
            for half in (128, 64, 32, 16, 8, 4, 2, 1):
                be = plsc.load_gather(eb_v, [ce + (half - 1)])
                bp = plsc.load_gather(pb_v, [cp + (half - 1)])
                ce = jnp.where(be < ve, ce + half, ce)
                cp = jnp.where(bp < vp, cp + half, cp)
            ide_v[pl.ds(off, 16)] = ce
            idp_v[pl.ds(off, 16)] = cp
        return 0

    lax.fori_loop(0, n_iter, body, 0)
    pltpu.sync_copy(ide_v, ie_hbm.at[pl.ds(base, rows_per_w)])
    pltpu.sync_copy(idp_v, ip_hbm.at[pl.ds(base, rows_per_w)])


def _sc_bucketize(et_flat, pt_flat, eb, pb):
    n = et_flat.shape[0]
    info = plsc.get_sparse_core_info()
    nc, ns = info.num_cores, info.num_subcores
    rows_per_w = n // (nc * ns)
    mesh = plsc.VectorSubcoreMesh(core_axis_name="c", subcore_axis_name="s")
    fn = functools.partial(
        pl.kernel,
        mesh=mesh,
        out_type=[jax.ShapeDtypeStruct((n,), jnp.int32),
                  jax.ShapeDtypeStruct((n,), jnp.int32)],
        scratch_types=[
            pltpu.VMEM((rows_per_w,), jnp.float32),
            pltpu.VMEM((rows_per_w,), jnp.float32),
            pltpu.VMEM((rows_per_w,), jnp.int32),
            pltpu.VMEM((rows_per_w,), jnp.int32),
            pltpu.VMEM((N_BOUNDS,), jnp.float32),
            pltpu.VMEM((N_BOUNDS,), jnp.float32),
        ],
        compiler_params=pltpu.CompilerParams(needs_layout_passes=False),
    )(functools.partial(_sc_bucketize_body, rows_per_w=rows_per_w,
                        num_cores=nc))
    return fn(et_flat, pt_flat, eb, pb)


def _tc_body(ie_ref, ip_ref, x_ref, tab_ref, out_ref):
    ie = ie_ref[0]
    ip = ip_ref[0] + N_BINS
    rows = lax.broadcasted_iota(jnp.int32, (2 * N_BINS, ROWS_BLOCK), 0)
    onehot_t = ((rows == ie) | (rows == ip)).astype(jnp.bfloat16)
    emb = lax.dot_general(onehot_t, tab_ref[...],
                          dimension_numbers=(((0,), (0,)), ((), ())),
                          preferred_element_type=jnp.float32)
    out_ref[...] = x_ref[...] + emb


def _tc_combine(x2d, ie3, ip3, tab):
    n = x2d.shape[0]
    grid = n // ROWS_BLOCK
    return pl.pallas_call(
        _tc_body,
        grid=(grid,),
        in_specs=[
            pl.BlockSpec((1, 1, ROWS_BLOCK), lambda i: (i, 0, 0)),
            pl.BlockSpec((1, 1, ROWS_BLOCK), lambda i: (i, 0, 0)),
            pl.BlockSpec((ROWS_BLOCK, D), lambda i: (i, 0)),
            pl.BlockSpec((2 * N_BINS, D), lambda i: (0, 0)),
        ],
        out_specs=pl.BlockSpec((ROWS_BLOCK, D), lambda i: (i, 0)),
        out_shape=jax.ShapeDtypeStruct((n, D), jnp.float32),
        compiler_params=pltpu.CompilerParams(
            fuse_transposed_lhs_in_matmul=True),
    )(ie3, ip3, x2d, tab)


def kernel(x, energy_target, pitch_target, energy_boundaries,
           pitch_boundaries, energy_table, pitch_table):
    b, t, d = x.shape
    n = b * t
    et = energy_target.reshape(n)
    pt = pitch_target.reshape(n)

    ie, ip = _sc_bucketize(et, pt, energy_boundaries, pitch_boundaries)

    tab = jnp.concatenate([energy_table, pitch_table], axis=0).astype(jnp.bfloat16)
    g = n // ROWS_BLOCK
    out2d = _tc_combine(x.reshape(n, d), ie.reshape(g, 1, ROWS_BLOCK),
                        ip.reshape(g, 1, ROWS_BLOCK), tab)
    return out2d.reshape(b, t, d)


# speedup vs baseline: 1.0249x; 1.0249x over previous
import functools

import jax
import jax.numpy as jnp
from jax import lax
from jax.experimental import pallas as pl
from jax.experimental.pallas import tpu as pltpu
from jax.experimental.pallas import tpu_sc as plsc

N_BINS = 256
N_BOUNDS = 255
D = 512
ROWS_BLOCK = 1024


def _sc_bucketize_body(et_hbm, pt_hbm, eb_hbm, pb_hbm, ie_hbm, ip_hbm,
                       te_v, tp_v, ide_v, idp_v, eb_v, pb_v, sem,
                       *, rows_per_w, num_cores):
    wid = lax.axis_index("s") * num_cores + lax.axis_index("c")
    base = wid * rows_per_w
    copies = [
        pltpu.async_copy(eb_hbm, eb_v, sem),
        pltpu.async_copy(pb_hbm, pb_v, sem),
        pltpu.async_copy(et_hbm.at[pl.ds(base, rows_per_w)], te_v, sem),
        pltpu.async_copy(pt_hbm.at[pl.ds(base, rows_per_w)], tp_v, sem),
    ]
    for c in copies:
        c.wait()

    unroll = 4
    n_iter = rows_per_w // (16 * unroll)

    def body(i, _):
        for u in range(unroll):
            off = (i * unroll + u) * 16
            ve = te_v[pl.ds(off, 16)]
            vp = tp_v[pl.ds(off, 16)]
            ce = jnp.zeros((16,), jnp.int32)
            cp = jnp.zeros((16,), jnp.int32)
            for half in (128, 64, 32, 16, 8, 4, 2, 1):
                be = plsc.load_gather(eb_v, [ce + (half - 1)])
                bp = plsc.load_gather(pb_v, [cp + (half - 1)])
                ce = jnp.where(be < ve, ce + half, ce)
                cp = jnp.where(bp < vp, cp + half, cp)
            ide_v[pl.ds(off, 16)] = ce
            idp_v[pl.ds(off, 16)] = cp
        return 0

    lax.fori_loop(0, n_iter, body, 0)
    w1 = pltpu.async_copy(ide_v, ie_hbm.at[pl.ds(base, rows_per_w)], sem)
    w2 = pltpu.async_copy(idp_v, ip_hbm.at[pl.ds(base, rows_per_w)], sem)
    w1.wait()
    w2.wait()


def _sc_bucketize(et_flat, pt_flat, eb, pb):
    n = et_flat.shape[0]
    info = plsc.get_sparse_core_info()
    nc, ns = info.num_cores, info.num_subcores
    rows_per_w = n // (nc * ns)
    mesh = plsc.VectorSubcoreMesh(core_axis_name="c", subcore_axis_name="s")
    fn = functools.partial(
        pl.kernel,
        mesh=mesh,
        out_type=[jax.ShapeDtypeStruct((n,), jnp.int32),
                  jax.ShapeDtypeStruct((n,), jnp.int32)],
        scratch_types=[
            pltpu.VMEM((rows_per_w,), jnp.float32),
            pltpu.VMEM((rows_per_w,), jnp.float32),
            pltpu.VMEM((rows_per_w,), jnp.int32),
            pltpu.VMEM((rows_per_w,), jnp.int32),
            pltpu.VMEM((N_BOUNDS,), jnp.float32),
            pltpu.VMEM((N_BOUNDS,), jnp.float32),
            pltpu.SemaphoreType.DMA,
        ],
        compiler_params=pltpu.CompilerParams(needs_layout_passes=False),
    )(functools.partial(_sc_bucketize_body, rows_per_w=rows_per_w,
                        num_cores=nc))
    return fn(et_flat, pt_flat, eb, pb)


def _tc_body(ie_ref, ip_ref, x_ref, tab_ref, out_ref):
    ie = ie_ref[0].reshape(1, ROWS_BLOCK)
    ip = ip_ref[0].reshape(1, ROWS_BLOCK) + N_BINS
    rows = lax.broadcasted_iota(jnp.int32, (2 * N_BINS, ROWS_BLOCK), 0)
    onehot_t = ((rows == ie) | (rows == ip)).astype(jnp.bfloat16)
    emb = lax.dot_general(onehot_t, tab_ref[...],
                          dimension_numbers=(((0,), (0,)), ((), ())),
                          preferred_element_type=jnp.float32)
    out_ref[...] = x_ref[...] + emb


def _tc_combine(x2d, ie3, ip3, tab):
    n = x2d.shape[0]
    grid = n // ROWS_BLOCK
    return pl.pallas_call(
        _tc_body,
        grid=(grid,),
        in_specs=[
            pl.BlockSpec((1, 8, 128), lambda i: (i, 0, 0)),
            pl.BlockSpec((1, 8, 128), lambda i: (i, 0, 0)),
            pl.BlockSpec((ROWS_BLOCK, D), lambda i: (i, 0)),
            pl.BlockSpec((2 * N_BINS, D), lambda i: (0, 0)),
        ],
        out_specs=pl.BlockSpec((ROWS_BLOCK, D), lambda i: (i, 0)),
        out_shape=jax.ShapeDtypeStruct((n, D), jnp.float32),
        compiler_params=pltpu.CompilerParams(
            fuse_transposed_lhs_in_matmul=True),
    )(ie3, ip3, x2d, tab)


def kernel(x, energy_target, pitch_target, energy_boundaries,
           pitch_boundaries, energy_table, pitch_table):
    b, t, d = x.shape
    n = b * t
    et = energy_target.reshape(n)
    pt = pitch_target.reshape(n)

    ie, ip = _sc_bucketize(et, pt, energy_boundaries, pitch_boundaries)

    tab = jnp.concatenate([energy_table, pitch_table], axis=0).astype(jnp.bfloat16)
    g = n // ROWS_BLOCK
    out2d = _tc_combine(x.reshape(n, d), ie.reshape(g, 8, 128),
                        ip.reshape(g, 8, 128), tab)
    return out2d.reshape(b, t, d)
